# baseline (speedup 1.0000x reference)
import functools

import jax
import jax.numpy as jnp
from jax import lax
from jax.experimental import pallas as pl
from jax.experimental.pallas import tpu as pltpu
from jax.experimental.pallas import tpu_sc as plsc


def _proj_body(tab_ref, w_ref, b_ref, out_ref):
    out_ref[...] = (lax.dot_general(
        w_ref[...], tab_ref[...], (((1,), (0,)), ((), ())),
        preferred_element_type=jnp.float32) + b_ref[0, 0])[0]


def _project(table, fc_w, fc_b, blkv):
    vocab, d = table.shape
    n = -(-vocab // blkv)
    out = pl.pallas_call(
        _proj_body,
        grid=(n,),
        in_specs=[
            pl.BlockSpec((d, blkv), lambda i: (0, i)),
            pl.BlockSpec((1, d), lambda i: (0, 0)),
            pl.BlockSpec(memory_space=pltpu.SMEM),
        ],
        out_specs=pl.BlockSpec((blkv,), lambda i: (i,)),
        out_shape=jax.ShapeDtypeStruct((n * blkv,), jnp.float32),
    )(table.T, fc_w, fc_b.reshape(1, 1))
    return out


def _make_sc_pool(l_seq, batch, proj_len):
    info = plsc.get_sparse_core_info()
    nc, ns, lanes = info.num_cores, info.num_subcores, info.num_lanes
    nw = nc * ns
    bpw = batch // nw
    groups = bpw // lanes
    lookahead = 24
    mesh = plsc.VectorSubcoreMesh(core_axis_name="c", subcore_axis_name="s")

    @functools.partial(
        pl.kernel,
        mesh=mesh,
        out_type=jax.ShapeDtypeStruct((batch,), jnp.float32),
        scratch_types=[
            pltpu.VMEM((l_seq, bpw), jnp.int32),
            pltpu.VMEM((l_seq, bpw), jnp.float32),
            pltpu.VMEM((bpw,), jnp.float32),
            pltpu.VMEM_SHARED((proj_len,), jnp.float32),
            pltpu.SemaphoreType.DMA,
        ],
    )
    def sc_pool(x_hbm, proj_hbm, out_hbm, idx_v, vals_v, out_v, proj_s, sem):
        wid = lax.axis_index("s") * nc + lax.axis_index("c")
        base = wid * bpw
        sid = lax.axis_index("s")
        slc = proj_len // ns
        pltpu.sync_copy(proj_hbm.at[pl.ds(sid * slc, slc)],
                        proj_s.at[pl.ds(sid * slc, slc)])
        plsc.subcore_barrier()
        pltpu.sync_copy(x_hbm.at[:, pl.ds(base, bpw)], idx_v)

        def fire(l):
            pltpu.async_copy(proj_s.at[idx_v.at[l]], vals_v.at[l], sem)

        def drain(l):
            pltpu.make_async_copy(
                proj_s.at[idx_v.at[l]], vals_v.at[l], sem).wait()

        def warm(l, c):
            fire(l)
            return c
        lax.fori_loop(0, lookahead, warm, 0)

        def steady(l, c):
            fire(l)
            drain(l - lookahead)
            return c
        lax.fori_loop(lookahead, l_seq, steady, 0)

        def tail(l, c):
            drain(l)
            return c
        lax.fori_loop(l_seq - lookahead, l_seq, tail, 0)

        def red(l, accs):
            return tuple(accs[g] + vals_v[l, pl.ds(g * lanes, lanes)]
                         for g in range(groups))
        accs = lax.fori_loop(
            0, l_seq, red,
            tuple(jnp.zeros((lanes,), jnp.float32) for _ in range(groups)))

        inv = jnp.float32(1.0 / l_seq)
        for g in range(groups):
            z = accs[g] * inv
            out_v[pl.ds(g * lanes, lanes)] = 1.0 / (1.0 + jnp.exp(-z))
        pltpu.sync_copy(out_v, out_hbm.at[pl.ds(base, bpw)])

    return sc_pool


def kernel(x, table, fc_w, fc_b):
    l_seq, batch = x.shape
    proj = _project(table, fc_w, fc_b, blkv=16384)
    sc_pool = _make_sc_pool(l_seq, batch, proj.shape[0])
    return sc_pool(x.astype(jnp.int32), proj)


# speedup vs baseline: 1.0801x; 1.0801x over previous
import functools

import jax
import jax.numpy as jnp
from jax import lax
from jax.experimental import pallas as pl
from jax.experimental.pallas import tpu as pltpu
from jax.experimental.pallas import tpu_sc as plsc


def _proj_body(tab_ref, w_ref, b_ref, out_ref):
    out_ref[...] = (lax.dot_general(
        w_ref[...], tab_ref[...], (((1,), (0,)), ((), ())),
        preferred_element_type=jnp.float32) + b_ref[0, 0])[0]


def _project(table, fc_w, fc_b, blkv):
    vocab, d = table.shape
    n = -(-vocab // blkv)
    out = pl.pallas_call(
        _proj_body,
        grid=(n,),
        in_specs=[
            pl.BlockSpec((d, blkv), lambda i: (0, i)),
            pl.BlockSpec((1, d), lambda i: (0, 0)),
            pl.BlockSpec(memory_space=pltpu.SMEM),
        ],
        out_specs=pl.BlockSpec((blkv,), lambda i: (i,)),
        out_shape=jax.ShapeDtypeStruct((n * blkv,), jnp.float32),
    )(table.T, fc_w, fc_b.reshape(1, 1))
    return out


def _make_sc_pool(l_seq, batch, proj_len):
    info = plsc.get_sparse_core_info()
    nc, ns, lanes = info.num_cores, info.num_subcores, info.num_lanes
    nw = nc * ns
    bpw = batch // nw
    groups = bpw // lanes
    lookahead = 24
    mesh = plsc.VectorSubcoreMesh(core_axis_name="c", subcore_axis_name="s")

    @functools.partial(
        pl.kernel,
        mesh=mesh,
        out_type=jax.ShapeDtypeStruct((batch,), jnp.float32),
        scratch_types=[
            pltpu.VMEM((l_seq, bpw), jnp.int32),
            pltpu.VMEM((l_seq, bpw), jnp.float32),
            pltpu.VMEM((bpw,), jnp.float32),
            pltpu.VMEM_SHARED((proj_len,), jnp.float32),
            pltpu.SemaphoreType.DMA,
        ],
    )
    def sc_pool(x_hbm, proj_hbm, out_hbm, idx_v, vals_v, out_v, proj_s, sem):
        wid = lax.axis_index("s") * nc + lax.axis_index("c")
        base = wid * bpw
        sid = lax.axis_index("s")
        slc = proj_len // ns
        pltpu.sync_copy(proj_hbm.at[pl.ds(sid * slc, slc)],
                        proj_s.at[pl.ds(sid * slc, slc)])
        plsc.subcore_barrier()
        pltpu.sync_copy(x_hbm.at[:, pl.ds(base, bpw)], idx_v)

        def fire(l):
            pltpu.async_copy(proj_s.at[idx_v.at[l]], vals_v.at[l], sem)

        def drain(l):
            pltpu.make_async_copy(
                proj_s.at[idx_v.at[l]], vals_v.at[l], sem).wait()

        def warm(l, c):
            fire(l)
            return c
        lax.fori_loop(0, lookahead, warm, 0)

        def steady(l, c):
            fire(l)
            drain(l - lookahead)
            return c
        lax.fori_loop(lookahead, l_seq, steady, 0)

        def tail(l, c):
            drain(l)
            return c
        lax.fori_loop(l_seq - lookahead, l_seq, tail, 0)

        def red(l, accs):
            return tuple(accs[g] + vals_v[l, pl.ds(g * lanes, lanes)]
                         for g in range(groups))
        accs = lax.fori_loop(
            0, l_seq, red,
            tuple(jnp.zeros((lanes,), jnp.float32) for _ in range(groups)))

        inv = jnp.float32(1.0 / l_seq)
        for g in range(groups):
            z = accs[g] * inv
            out_v[pl.ds(g * lanes, lanes)] = 1.0 / (1.0 + jnp.exp(-z))
        pltpu.sync_copy(out_v, out_hbm.at[pl.ds(base, bpw)])

    return sc_pool


def kernel(x, table, fc_w, fc_b):
    l_seq, batch = x.shape
    proj = _project(table, fc_w, fc_b, blkv=49152)
    sc_pool = _make_sc_pool(l_seq, batch, proj.shape[0])
    return sc_pool(x.astype(jnp.int32), proj)


# speedup vs baseline: 1.0997x; 1.0182x over previous
import functools

import jax
import jax.numpy as jnp
from jax import lax
from jax.experimental import pallas as pl
from jax.experimental.pallas import tpu as pltpu
from jax.experimental.pallas import tpu_sc as plsc


def _proj_body(tab_ref, w_ref, b_ref, out_ref):
    out_ref[...] = (lax.dot_general(
        w_ref[...], tab_ref[...], (((1,), (0,)), ((), ())),
        preferred_element_type=jnp.float32) + b_ref[0, 0])[0]


def _project(table, fc_w, fc_b, blkv):
    vocab, d = table.shape
    n = -(-vocab // blkv)
    out = pl.pallas_call(
        _proj_body,
        grid=(n,),
        in_specs=[
            pl.BlockSpec((d, blkv), lambda i: (0, i)),
            pl.BlockSpec((1, d), lambda i: (0, 0)),
            pl.BlockSpec(memory_space=pltpu.SMEM),
        ],
        out_specs=pl.BlockSpec((blkv,), lambda i: (i,)),
        out_shape=jax.ShapeDtypeStruct((n * blkv,), jnp.float32),
    )(table.T, fc_w, fc_b.reshape(1, 1))
    return out


def _make_sc_pool(l_seq, batch, proj_len):
    info = plsc.get_sparse_core_info()
    nc, ns, lanes = info.num_cores, info.num_subcores, info.num_lanes
    nw = nc * ns
    bpw = batch // nw
    groups = bpw // lanes
    mesh = plsc.VectorSubcoreMesh(core_axis_name="c", subcore_axis_name="s")

    @functools.partial(
        pl.kernel,
        mesh=mesh,
        out_type=jax.ShapeDtypeStruct((batch,), jnp.float32),
        scratch_types=[
            pltpu.VMEM((l_seq, bpw), jnp.int32),
            pltpu.VMEM((l_seq * bpw,), jnp.float32),
            pltpu.VMEM((bpw,), jnp.float32),
            pltpu.VMEM_SHARED((proj_len,), jnp.float32),
            pltpu.SemaphoreType.DMA,
            pltpu.SemaphoreType.DMA,
        ],
    )
    def sc_pool(x_hbm, proj_hbm, out_hbm, idx_v, vals_v, out_v, proj_s,
                sem, stage_sem):
        wid = lax.axis_index("s") * nc + lax.axis_index("c")
        base = wid * bpw
        sid = lax.axis_index("s")
        slc = proj_len // ns
        stage = pltpu.make_async_copy(proj_hbm.at[pl.ds(sid * slc, slc)],
                                      proj_s.at[pl.ds(sid * slc, slc)],
                                      stage_sem)
        stage.start()
        pltpu.sync_copy(x_hbm.at[:, pl.ds(base, bpw)], idx_v)
        stage.wait()
        plsc.subcore_barrier()

        def fire(l4, c):
            for u in range(4):
                l = l4 * 4 + u
                pltpu.async_copy(
                    proj_s.at[idx_v.at[l]],
                    vals_v.at[pl.ds(l * bpw, bpw)], sem)
            return c
        lax.fori_loop(0, l_seq // 4, fire, 0)

        pltpu.make_async_copy(
            proj_hbm.at[pl.ds(0, l_seq * bpw)], vals_v, sem).wait()

        def red(l, accs):
            return tuple(
                accs[g] + vals_v[pl.ds(l * bpw + g * lanes, lanes)]
                for g in range(groups))
        accs = lax.fori_loop(
            0, l_seq, red,
            tuple(jnp.zeros((lanes,), jnp.float32) for _ in range(groups)))

        inv = jnp.float32(1.0 / l_seq)
        for g in range(groups):
            z = accs[g] * inv
            out_v[pl.ds(g * lanes, lanes)] = 1.0 / (1.0 + jnp.exp(-z))
        pltpu.sync_copy(out_v, out_hbm.at[pl.ds(base, bpw)])

    return sc_pool


def kernel(x, table, fc_w, fc_b):
    l_seq, batch = x.shape
    proj = _project(table, fc_w, fc_b, blkv=36864)
    sc_pool = _make_sc_pool(l_seq, batch, proj.shape[0])
    return sc_pool(x.astype(jnp.int32), proj)
